# baseline (speedup 1.0000x reference)
import functools
import jax
import jax.numpy as jnp
import numpy as np
from jax import lax
from jax.experimental import pallas as pl
from jax.experimental.pallas import tpu as pltpu
from jax.experimental.pallas import tpu_sc as plsc

_B = 16384
_D = 128

_NC = 2
_NS = 16
_NW = _NC * _NS

_C = 64
_NSPLIT = 2
_BS = _B // _NSPLIT
_BPW = _BS // _NW
_NK = _BPW // _C

_PACK_PERM = np.array(
    [32 * g + (r // 2) + 16 * (r % 2) for g in range(_D // 32)
     for r in range(32)], dtype=np.int32)


def _pack_rows_chunk(src_ref, dst_ref):
    def row(r, _):
        for g in range(_D // 32):
            p0 = src_ref[r, pl.ds(32 * g, 16)]
            p1 = src_ref[r, pl.ds(32 * g + 16, 16)]
            dst_ref[r, pl.ds(32 * g, 32)] = plsc.pack(
                p0, p1, format=plsc.PackFormat.INTERLEAVED)
        return 0
    lax.fori_loop(0, _C, row, 0)


def _product_pack_chunk(a_ref, b_ref, dst_ref):
    def row(r, _):
        for g in range(_D // 32):
            s0 = pl.ds(32 * g, 16)
            s1 = pl.ds(32 * g + 16, 16)
            p0 = a_ref[r, s0] * b_ref[r, s0]
            p1 = a_ref[r, s1] * b_ref[r, s1]
            dst_ref[r, pl.ds(32 * g, 32)] = plsc.pack(
                p0, p1, format=plsc.PackFormat.INTERLEAVED)
        return 0
    lax.fori_loop(0, _C, row, 0)


def _sc_body(off, user_hbm, item_hbm, ug_hbm, ig_hbm, um_hbm, im_hbm,
             out_gmf, out_mu, out_mi,
             uidx_v, iidx_v, *rest):
    A = rest[0:2]
    Bb = rest[2:4]
    MU = rest[4:6]
    MI = rest[6:8]
    GB = rest[8:10]
    MUB = rest[10:12]
    MIB = rest[12:14]
    gsem = rest[14:22]
    wsem = rest[22:28]
    wG = wsem[0:2]
    wMU = wsem[2:4]
    wMI = wsem[4:6]

    wid = lax.axis_index("s") * _NC + lax.axis_index("c")
    base = wid * _BPW
    pltpu.sync_copy(user_hbm.at[pl.ds(off + base, _BPW)], uidx_v)
    pltpu.sync_copy(item_hbm.at[pl.ds(off + base, _BPW)], iidx_v)

    gh = [None] * _NK
    wb = {}

    def fire_gathers(k):
        s = k % 2
        for key in (("G", s), ("MU", s), ("MI", s)):
            if key in wb:
                wb.pop(key).wait()
        co = k * _C
        uidx = uidx_v.at[pl.ds(co, _C)]
        iidx = iidx_v.at[pl.ds(co, _C)]
        gh[k] = (
            pltpu.async_copy(ug_hbm.at[uidx], A[s], gsem[0 + s]),
            pltpu.async_copy(ig_hbm.at[iidx], Bb[s], gsem[2 + s]),
            pltpu.async_copy(um_hbm.at[uidx], MU[s], gsem[4 + s]),
            pltpu.async_copy(im_hbm.at[iidx], MI[s], gsem[6 + s]),
        )

    def drain_chunk(j):
        s = j % 2
        dst = pl.ds(off + base + j * _C, _C)
        ga, gb, gmu, gmi = gh[j]
        ga.wait()
        gb.wait()
        _product_pack_chunk(A[s], Bb[s], GB[s])
        wb[("G", s)] = pltpu.make_async_copy(GB[s], out_gmf.at[dst], wG[s])
        wb[("G", s)].start()
        gmu.wait()
        _pack_rows_chunk(MU[s], MUB[s])
        wb[("MU", s)] = pltpu.make_async_copy(MUB[s], out_mu.at[dst], wMU[s])
        wb[("MU", s)].start()
        gmi.wait()
        _pack_rows_chunk(MI[s], MIB[s])
        wb[("MI", s)] = pltpu.make_async_copy(MIB[s], out_mi.at[dst], wMI[s])
        wb[("MI", s)].start()

    fire_gathers(0)
    for k in range(1, _NK + 1):
        if k < _NK:
            fire_gathers(k)
        drain_chunk(k - 1)
    for h in wb.values():
        h.wait()


@functools.cache
def _sc_gather(off):
    return pl.kernel(
        functools.partial(_sc_body, off),
        out_type=[jax.ShapeDtypeStruct((_B, _D), jnp.bfloat16)] * 3,
        mesh=plsc.VectorSubcoreMesh(core_axis_name="c", subcore_axis_name="s"),
        compiler_params=pltpu.CompilerParams(needs_layout_passes=False),
        scratch_types=(
            [pltpu.VMEM((_BPW,), jnp.int32),
             pltpu.VMEM((_BPW,), jnp.int32)]
            + [pltpu.VMEM((_C, _D), jnp.float32)] * 8
            + [pltpu.VMEM((_C, _D), jnp.bfloat16)] * 6
            + [pltpu.SemaphoreType.DMA] * 14
        ),
    )


_BLK = 1024


def _dense_body(gmf_ref, mu_ref, mi_ref,
                w1a_ref, w1b_ref, b1_ref, w2_ref, b2_ref, w3_ref, b3_ref,
                wog_ref, woh_ref, bo_ref, out_ref):
    h = jnp.maximum(
        jnp.dot(mu_ref[...], w1a_ref[...], preferred_element_type=jnp.float32)
        + jnp.dot(mi_ref[...], w1b_ref[...], preferred_element_type=jnp.float32)
        + b1_ref[...], 0.0)
    h = jnp.maximum(
        jnp.dot(h, w2_ref[...], preferred_element_type=jnp.float32)
        + b2_ref[...], 0.0)
    h = jnp.maximum(
        jnp.dot(h, w3_ref[...], preferred_element_type=jnp.float32)
        + b3_ref[...], 0.0)
    rnd = lambda x: x.astype(jnp.bfloat16).astype(jnp.float32)
    logits = (jnp.sum(gmf_ref[...].astype(jnp.float32)
                      * wog_ref[...].astype(jnp.float32), axis=1)
              + jnp.sum(rnd(h) * rnd(woh_ref[...]), axis=1)
              + bo_ref[0, 0])
    out_ref[...] = logits


def _dense(off, gmf, mu, mi, w1a, w1b, b1r, W2, b2r, W3, b3r, wog, woh, bor):
    grid = _BS // _BLK
    ob = off // _BLK
    blk_in = pl.BlockSpec((_BLK, _D), lambda i: (i + ob, 0))
    rep = lambda shape: pl.BlockSpec(shape, lambda i: tuple(0 for _ in shape))
    return pl.pallas_call(
        _dense_body,
        grid=(grid,),
        in_specs=[blk_in, blk_in, blk_in,
                  rep(w1a.shape), rep(w1b.shape), rep(b1r.shape),
                  rep(W2.shape), rep(b2r.shape),
                  rep(W3.shape), rep(b3r.shape),
                  rep(wog.shape), rep(woh.shape), rep(bor.shape)],
        out_specs=pl.BlockSpec((_BLK,), lambda i: (i,)),
        out_shape=jax.ShapeDtypeStruct((_BS,), jnp.float32),
    )(gmf, mu, mi, w1a, w1b, b1r, W2, b2r, W3, b3r, wog, woh, bor)


@jax.jit
def kernel(user, item, ue_gmf, ie_gmf, ue_mlp, ie_mlp,
           W1, b1, W2, b2, W3, b3, Wo, bo):
    perm = jnp.asarray(_PACK_PERM)
    w1a = W1[:_D][perm].astype(jnp.bfloat16)
    w1b = W1[_D:][perm].astype(jnp.bfloat16)
    wog = Wo[:_D, 0][perm].reshape(1, _D).astype(jnp.bfloat16)
    woh = Wo[_D:, 0].reshape(1, -1)
    b1r = b1.reshape(1, -1)
    b2r = b2.reshape(1, -1)
    b3r = b3.reshape(1, -1)
    bor = bo.reshape(1, 1)
    outs = []
    for s in range(_NSPLIT):
        off = s * _BS
        gmf, mu, mi = _sc_gather(off)(user, item, ue_gmf, ie_gmf,
                                      ue_mlp, ie_mlp)
        outs.append(_dense(off, gmf, mu, mi, w1a, w1b, b1r, W2, b2r,
                           W3, b3r, wog, woh, bor))
    return jnp.concatenate(outs)


# speedup vs baseline: 1.2107x; 1.2107x over previous
import functools
import jax
import jax.numpy as jnp
import numpy as np
from jax import lax
from jax.experimental import pallas as pl
from jax.experimental.pallas import tpu as pltpu
from jax.experimental.pallas import tpu_sc as plsc

_B = 16384
_D = 128

_NC = 2
_NS = 16
_NW = _NC * _NS

_C = 64
_NSPLIT = 2
_BS = _B // _NSPLIT
_BPW = _BS // _NW
_NK = _BPW // _C

_PACK_PERM = np.array(
    [32 * g + (r // 2) + 16 * (r % 2) for g in range(_D // 32)
     for r in range(32)], dtype=np.int32)


def _product_pack_chunk(a_ref, b_ref, dst_ref):
    def row(r, _):
        for g in range(_D // 32):
            s0 = pl.ds(32 * g, 16)
            s1 = pl.ds(32 * g + 16, 16)
            p0 = a_ref[r, s0] * b_ref[r, s0]
            p1 = a_ref[r, s1] * b_ref[r, s1]
            dst_ref[r, pl.ds(32 * g, 32)] = plsc.pack(
                p0, p1, format=plsc.PackFormat.INTERLEAVED)
        return 0
    lax.fori_loop(0, _C, row, 0)


def _sc_body(off, user_hbm, item_hbm, ug_hbm, ig_hbm, um_hbm, im_hbm,
             out_gmf, out_mu, out_mi,
             uidx_v, iidx_v, *rest):
    A = rest[0:2]
    Bb = rest[2:4]
    MU = rest[4:6]
    MI = rest[6:8]
    GB = rest[8:10]
    gsem = rest[10:18]
    wsem = rest[18:24]
    wG = wsem[0:2]
    wMU = wsem[2:4]
    wMI = wsem[4:6]

    wid = lax.axis_index("s") * _NC + lax.axis_index("c")
    base = wid * _BPW
    pltpu.sync_copy(user_hbm.at[pl.ds(off + base, _BPW)], uidx_v)
    pltpu.sync_copy(item_hbm.at[pl.ds(off + base, _BPW)], iidx_v)

    gh = [None] * _NK
    wb = {}

    def fire_gathers(k):
        s = k % 2
        for key in (("G", s), ("MU", s), ("MI", s)):
            if key in wb:
                wb.pop(key).wait()
        co = k * _C
        uidx = uidx_v.at[pl.ds(co, _C)]
        iidx = iidx_v.at[pl.ds(co, _C)]
        gh[k] = (
            pltpu.async_copy(ug_hbm.at[uidx], A[s], gsem[0 + s]),
            pltpu.async_copy(ig_hbm.at[iidx], Bb[s], gsem[2 + s]),
            pltpu.async_copy(um_hbm.at[uidx], MU[s], gsem[4 + s]),
            pltpu.async_copy(im_hbm.at[iidx], MI[s], gsem[6 + s]),
        )

    def drain_chunk(j):
        s = j % 2
        dst = pl.ds(off + base + j * _C, _C)
        ga, gb, gmu, gmi = gh[j]
        ga.wait()
        gb.wait()
        _product_pack_chunk(A[s], Bb[s], GB[s])
        wb[("G", s)] = pltpu.make_async_copy(GB[s], out_gmf.at[dst], wG[s])
        wb[("G", s)].start()
        gmu.wait()
        wb[("MU", s)] = pltpu.make_async_copy(MU[s], out_mu.at[dst], wMU[s])
        wb[("MU", s)].start()
        gmi.wait()
        wb[("MI", s)] = pltpu.make_async_copy(MI[s], out_mi.at[dst], wMI[s])
        wb[("MI", s)].start()

    fire_gathers(0)
    for k in range(1, _NK + 1):
        if k < _NK:
            fire_gathers(k)
        drain_chunk(k - 1)
    for h in wb.values():
        h.wait()


@functools.cache
def _sc_gather(off):
    return pl.kernel(
        functools.partial(_sc_body, off),
        out_type=[jax.ShapeDtypeStruct((_B, _D), jnp.bfloat16),
                  jax.ShapeDtypeStruct((_B, _D), jnp.float32),
                  jax.ShapeDtypeStruct((_B, _D), jnp.float32)],
        mesh=plsc.VectorSubcoreMesh(core_axis_name="c", subcore_axis_name="s"),
        compiler_params=pltpu.CompilerParams(needs_layout_passes=False),
        scratch_types=(
            [pltpu.VMEM((_BPW,), jnp.int32),
             pltpu.VMEM((_BPW,), jnp.int32)]
            + [pltpu.VMEM((_C, _D), jnp.float32)] * 8
            + [pltpu.VMEM((_C, _D), jnp.bfloat16)] * 2
            + [pltpu.SemaphoreType.DMA] * 14
        ),
    )


_BLK = 1024


def _dense_body(gmf_ref, mu_ref, mi_ref,
                w1a_ref, w1b_ref, b1_ref, w2_ref, b2_ref, w3_ref, b3_ref,
                wog_ref, woh_ref, bo_ref, out_ref):
    h = jnp.maximum(
        jnp.dot(mu_ref[...], w1a_ref[...], preferred_element_type=jnp.float32)
        + jnp.dot(mi_ref[...], w1b_ref[...], preferred_element_type=jnp.float32)
        + b1_ref[...], 0.0)
    h = jnp.maximum(
        jnp.dot(h, w2_ref[...], preferred_element_type=jnp.float32)
        + b2_ref[...], 0.0)
    h = jnp.maximum(
        jnp.dot(h, w3_ref[...], preferred_element_type=jnp.float32)
        + b3_ref[...], 0.0)
    rnd = lambda x: x.astype(jnp.bfloat16).astype(jnp.float32)
    logits = (jnp.sum(gmf_ref[...].astype(jnp.float32)
                      * wog_ref[...].astype(jnp.float32), axis=1)
              + jnp.sum(rnd(h) * rnd(woh_ref[...]), axis=1)
              + bo_ref[0, 0])
    out_ref[...] = logits


def _dense(off, gmf, mu, mi, w1a, w1b, b1r, W2, b2r, W3, b3r, wog, woh, bor):
    grid = _BS // _BLK
    ob = off // _BLK
    blk_in = pl.BlockSpec((_BLK, _D), lambda i: (i + ob, 0))
    rep = lambda shape: pl.BlockSpec(shape, lambda i: tuple(0 for _ in shape))
    return pl.pallas_call(
        _dense_body,
        grid=(grid,),
        in_specs=[blk_in, blk_in, blk_in,
                  rep(w1a.shape), rep(w1b.shape), rep(b1r.shape),
                  rep(W2.shape), rep(b2r.shape),
                  rep(W3.shape), rep(b3r.shape),
                  rep(wog.shape), rep(woh.shape), rep(bor.shape)],
        out_specs=pl.BlockSpec((_BLK,), lambda i: (i,)),
        out_shape=jax.ShapeDtypeStruct((_BS,), jnp.float32),
    )(gmf, mu, mi, w1a, w1b, b1r, W2, b2r, W3, b3r, wog, woh, bor)


@jax.jit
def kernel(user, item, ue_gmf, ie_gmf, ue_mlp, ie_mlp,
           W1, b1, W2, b2, W3, b3, Wo, bo):
    perm = jnp.asarray(_PACK_PERM)
    w1a = W1[:_D]
    w1b = W1[_D:]
    wog = Wo[:_D, 0][perm].reshape(1, _D).astype(jnp.bfloat16)
    woh = Wo[_D:, 0].reshape(1, -1)
    b1r = b1.reshape(1, -1)
    b2r = b2.reshape(1, -1)
    b3r = b3.reshape(1, -1)
    bor = bo.reshape(1, 1)
    outs = []
    for s in range(_NSPLIT):
        off = s * _BS
        gmf, mu, mi = _sc_gather(off)(user, item, ue_gmf, ie_gmf,
                                      ue_mlp, ie_mlp)
        outs.append(_dense(off, gmf, mu, mi, w1a, w1b, b1r, W2, b2r,
                           W3, b3r, wog, woh, bor))
    return jnp.concatenate(outs)


# speedup vs baseline: 1.2165x; 1.0048x over previous
import functools
import jax
import jax.numpy as jnp
import numpy as np
from jax import lax
from jax.experimental import pallas as pl
from jax.experimental.pallas import tpu as pltpu
from jax.experimental.pallas import tpu_sc as plsc

_B = 16384
_D = 128

_NC = 2
_NS = 16
_NW = _NC * _NS

_C = 64
_NSPLIT = 2
_BS = _B // _NSPLIT
_BPW = _BS // _NW
_NK = _BPW // _C

_PACK_PERM = np.array(
    [32 * g + (r // 2) + 16 * (r % 2) for g in range(_D // 32)
     for r in range(32)], dtype=np.int32)


def _product_pack_chunk(a_ref, b_ref, dst_ref):
    def row(r, _):
        for g in range(_D // 32):
            s0 = pl.ds(32 * g, 16)
            s1 = pl.ds(32 * g + 16, 16)
            p0 = a_ref[r, s0] * b_ref[r, s0]
            p1 = a_ref[r, s1] * b_ref[r, s1]
            dst_ref[r, pl.ds(32 * g, 32)] = plsc.pack(
                p0, p1, format=plsc.PackFormat.INTERLEAVED)
        return 0
    lax.fori_loop(0, _C, row, 0)


def _sc_body(off, user_hbm, item_hbm, ug_hbm, ig_hbm, um_hbm, im_hbm,
             out_gmf, out_mu, out_mi,
             uidx_v, iidx_v, *rest):
    A = rest[0:2]
    Bb = rest[2:4]
    MU = rest[4:6]
    MI = rest[6:8]
    GB = rest[8:10]
    gsem = rest[10:18]
    wsem = rest[18:24]
    wG = wsem[0:2]
    wMU = wsem[2:4]
    wMI = wsem[4:6]

    wid = lax.axis_index("s") * _NC + lax.axis_index("c")
    base = wid * _BPW
    pltpu.sync_copy(user_hbm.at[pl.ds(off + base, _BPW)], uidx_v)
    pltpu.sync_copy(item_hbm.at[pl.ds(off + base, _BPW)], iidx_v)

    gh = [None] * _NK
    wb = {}

    def fire_gathers(k):
        s = k % 2
        for key in (("G", s), ("MU", s), ("MI", s)):
            if key in wb:
                wb.pop(key).wait()
        co = k * _C
        uidx = uidx_v.at[pl.ds(co, _C)]
        iidx = iidx_v.at[pl.ds(co, _C)]
        gh[k] = (
            pltpu.async_copy(ug_hbm.at[uidx], A[s], gsem[0 + s]),
            pltpu.async_copy(ig_hbm.at[iidx], Bb[s], gsem[2 + s]),
            pltpu.async_copy(um_hbm.at[uidx], MU[s], gsem[4 + s]),
            pltpu.async_copy(im_hbm.at[iidx], MI[s], gsem[6 + s]),
        )

    def drain_chunk(j):
        s = j % 2
        dst = pl.ds(off + base + j * _C, _C)
        ga, gb, gmu, gmi = gh[j]
        ga.wait()
        gb.wait()
        _product_pack_chunk(A[s], Bb[s], GB[s])
        wb[("G", s)] = pltpu.make_async_copy(GB[s], out_gmf.at[dst], wG[s])
        wb[("G", s)].start()
        gmu.wait()
        wb[("MU", s)] = pltpu.make_async_copy(MU[s], out_mu.at[dst], wMU[s])
        wb[("MU", s)].start()
        gmi.wait()
        wb[("MI", s)] = pltpu.make_async_copy(MI[s], out_mi.at[dst], wMI[s])
        wb[("MI", s)].start()

    fire_gathers(0)
    for k in range(1, _NK + 1):
        if k < _NK:
            fire_gathers(k)
        drain_chunk(k - 1)
    for h in wb.values():
        h.wait()


@functools.cache
def _sc_gather(off):
    return pl.kernel(
        functools.partial(_sc_body, off),
        out_type=[jax.ShapeDtypeStruct((_B, _D), jnp.bfloat16),
                  jax.ShapeDtypeStruct((_B, _D), jnp.float32),
                  jax.ShapeDtypeStruct((_B, _D), jnp.float32)],
        mesh=plsc.VectorSubcoreMesh(core_axis_name="c", subcore_axis_name="s"),
        compiler_params=pltpu.CompilerParams(needs_layout_passes=False),
        scratch_types=(
            [pltpu.VMEM((_BPW,), jnp.int32),
             pltpu.VMEM((_BPW,), jnp.int32)]
            + [pltpu.VMEM((_C, _D), jnp.float32)] * 8
            + [pltpu.VMEM((_C, _D), jnp.bfloat16)] * 2
            + [pltpu.SemaphoreType.DMA] * 14
        ),
    )


_BLK = 1024


def _dense_body(gmf_ref, mu_ref, mi_ref,
                w1a_ref, w1b_ref, b1_ref, w2_ref, b2_ref, w3_ref, b3_ref,
                wog_ref, woh_ref, bo_ref, out_ref):
    mu_bf = mu_ref[...].astype(jnp.bfloat16)
    mi_bf = mi_ref[...].astype(jnp.bfloat16)
    h = jnp.maximum(
        jnp.dot(mu_bf, w1a_ref[...], preferred_element_type=jnp.float32)
        + jnp.dot(mi_bf, w1b_ref[...], preferred_element_type=jnp.float32)
        + b1_ref[...], 0.0)
    h = jnp.maximum(
        jnp.dot(h, w2_ref[...], preferred_element_type=jnp.float32)
        + b2_ref[...], 0.0)
    h = jnp.maximum(
        jnp.dot(h, w3_ref[...], preferred_element_type=jnp.float32)
        + b3_ref[...], 0.0)
    rnd = lambda x: x.astype(jnp.bfloat16).astype(jnp.float32)
    logits = (jnp.sum(gmf_ref[...].astype(jnp.float32)
                      * wog_ref[...].astype(jnp.float32), axis=1)
              + jnp.sum(rnd(h) * rnd(woh_ref[...]), axis=1)
              + bo_ref[0, 0])
    out_ref[...] = logits


def _dense(off, gmf, mu, mi, w1a, w1b, b1r, W2, b2r, W3, b3r, wog, woh, bor):
    grid = _BS // _BLK
    ob = off // _BLK
    blk_in = pl.BlockSpec((_BLK, _D), lambda i: (i + ob, 0))
    rep = lambda shape: pl.BlockSpec(shape, lambda i: tuple(0 for _ in shape))
    return pl.pallas_call(
        _dense_body,
        grid=(grid,),
        in_specs=[blk_in, blk_in, blk_in,
                  rep(w1a.shape), rep(w1b.shape), rep(b1r.shape),
                  rep(W2.shape), rep(b2r.shape),
                  rep(W3.shape), rep(b3r.shape),
                  rep(wog.shape), rep(woh.shape), rep(bor.shape)],
        out_specs=pl.BlockSpec((_BLK,), lambda i: (i,)),
        out_shape=jax.ShapeDtypeStruct((_BS,), jnp.float32),
    )(gmf, mu, mi, w1a, w1b, b1r, W2, b2r, W3, b3r, wog, woh, bor)


@jax.jit
def kernel(user, item, ue_gmf, ie_gmf, ue_mlp, ie_mlp,
           W1, b1, W2, b2, W3, b3, Wo, bo):
    perm = jnp.asarray(_PACK_PERM)
    w1a = W1[:_D].astype(jnp.bfloat16)
    w1b = W1[_D:].astype(jnp.bfloat16)
    wog = Wo[:_D, 0][perm].reshape(1, _D).astype(jnp.bfloat16)
    woh = Wo[_D:, 0].reshape(1, -1)
    b1r = b1.reshape(1, -1)
    b2r = b2.reshape(1, -1)
    b3r = b3.reshape(1, -1)
    bor = bo.reshape(1, 1)
    outs = []
    for s in range(_NSPLIT):
        off = s * _BS
        gmf, mu, mi = _sc_gather(off)(user, item, ue_gmf, ie_gmf,
                                      ue_mlp, ie_mlp)
        outs.append(_dense(off, gmf, mu, mi, w1a, w1b, b1r, W2, b2r,
                           W3, b3r, wog, woh, bor))
    return jnp.concatenate(outs)
